# Initial kernel scaffold; baseline (speedup 1.0000x reference)
import jax
import jax.numpy as jnp
from jax.experimental import pallas as pl


def kernel(X, batch, W0, b0, W1, b1, Wf, bf):
    raise NotImplementedError("write your pallas kernel here")


# speedup vs baseline: 7.6386x; 7.6386x over previous
import numpy as np
import jax
import jax.numpy as jnp
from jax.experimental import pallas as pl
from jax.experimental.pallas import tpu as pltpu

_N = 100000
_B = 2000
_NB = _N // _B
_G = 64
_BIG = np.int32(2**31 - 1)


def _threefry_bits(j):
    k0 = jnp.uint32(0)
    k1 = jnp.uint32(1234)
    ks2 = k0 ^ k1 ^ jnp.uint32(0x1BD11BDA)
    ks = [k0, k1, ks2]
    rots = [[13, 15, 26, 6], [17, 29, 16, 24]]
    x0 = jnp.zeros_like(j) + ks[0]
    x1 = j + ks[1]
    for i in range(5):
        for r in rots[i % 2]:
            x0 = x0 + x1
            x1 = (x1 << r) | (x1 >> (32 - r))
            x1 = x1 ^ x0
        x0 = x0 + ks[(i + 1) % 3]
        x1 = x1 + ks[(i + 2) % 3] + jnp.uint32(i + 1)
    return x0 ^ x1


def _gumbel_from_bits(bits):
    tiny = jnp.float32(np.finfo(np.float32).tiny)
    fb = (bits >> 9) | jnp.uint32(0x3F800000)
    f = jax.lax.bitcast_convert_type(fb, jnp.float32) - jnp.float32(1.0)
    u = jnp.maximum(tiny, f * (jnp.float32(1.0) - tiny) + tiny)
    return -jnp.log(-jnp.log(u))


def _body(xb_ref, bt_ref, w0_ref, b0_ref, w1_ref, b1_ref, wf_ref, bf_ref,
          p_ref, act_ref, a_ref,
          ssum, smax, sidx, sewin, sstart):
    step = pl.program_id(0)

    @pl.when(step == 0)
    def _init():
        ssum[...] = jnp.zeros_like(ssum)
        smax[...] = jnp.full_like(smax, -jnp.inf)
        sidx[...] = jnp.full_like(sidx, _BIG)
        sewin[...] = jnp.zeros_like(sewin)
        sstart[...] = jnp.full_like(sstart, _BIG)

    xb = xb_ref[...]
    b = bt_ref[0]

    h = jax.lax.dot_general(xb, w0_ref[...], (((1,), (0,)), ((), ())),
                            preferred_element_type=jnp.float32)
    h = jnp.maximum(h + b0_ref[...], jnp.float32(0.0))
    h = jax.lax.dot_general(h, w1_ref[...], (((1,), (0,)), ((), ())),
                            preferred_element_type=jnp.float32)
    h = jnp.maximum(h + b1_ref[...], jnp.float32(0.0))
    logits = jax.lax.dot_general(wf_ref[...], h, (((1,), (1,)), ((), ())),
                                 preferred_element_type=jnp.float32)
    logits = logits + bf_ref[...]
    e = jnp.exp(logits)

    i = step * _B + jax.lax.broadcasted_iota(jnp.int32, (1, _B), 1)
    j = ((63 - b) * _N + i).astype(jnp.uint32)
    score = jnp.log(e) + _gumbel_from_bits(_threefry_bits(j))

    gcol = 63 - jax.lax.broadcasted_iota(jnp.int32, (_G, 1), 0)
    m = b == gcol
    mf = m.astype(jnp.float32)

    ssum[...] += jnp.sum(mf * e, axis=1, keepdims=True)

    neg = jnp.float32(-jnp.inf)
    sm = jnp.where(m, score, neg)
    bmax = jnp.max(sm, axis=1, keepdims=True)
    bidx = jnp.min(jnp.where(m & (sm == bmax), i, _BIG), axis=1, keepdims=True)
    ew = jnp.sum(jnp.where(i == bidx, e, jnp.float32(0.0)) * mf,
                 axis=1, keepdims=True)

    upd = bmax > smax[...]
    smax[...] = jnp.where(upd, bmax, smax[...])
    sidx[...] = jnp.where(upd, bidx, sidx[...])
    sewin[...] = jnp.where(upd, ew, sewin[...])
    sstart[...] = jnp.minimum(
        sstart[...], jnp.min(jnp.where(m, i, _BIG), axis=1, keepdims=True))

    @pl.when(step == _NB - 1)
    def _fin():
        p_ref[...] = sewin[...] / ssum[...]
        a_ref[...] = sidx[...]
        act_ref[...] = sidx[...] - sstart[...]


def kernel(X, batch, W0, b0, W1, b1, Wf, bf):
    b2d = batch.astype(jnp.int32).reshape(_NB, 1, _B)
    b0r = b0.astype(jnp.float32).reshape(1, -1)
    b1r = b1.astype(jnp.float32).reshape(1, -1)
    wfr = Wf.astype(jnp.float32).reshape(-1, 1).T
    bfr = bf.astype(jnp.float32).reshape(1, 1)

    out_shapes = (
        jax.ShapeDtypeStruct((_G, 1), jnp.float32),
        jax.ShapeDtypeStruct((_G, 1), jnp.int32),
        jax.ShapeDtypeStruct((_G, 1), jnp.int32),
    )
    p, act, a = pl.pallas_call(
        _body,
        grid=(_NB,),
        in_specs=[
            pl.BlockSpec((_B, 128), lambda s: (s, 0)),
            pl.BlockSpec((1, 1, _B), lambda s: (s, 0, 0)),
            pl.BlockSpec((128, 64), lambda s: (0, 0)),
            pl.BlockSpec((1, 64), lambda s: (0, 0)),
            pl.BlockSpec((64, 64), lambda s: (0, 0)),
            pl.BlockSpec((1, 64), lambda s: (0, 0)),
            pl.BlockSpec((1, 64), lambda s: (0, 0)),
            pl.BlockSpec((1, 1), lambda s: (0, 0)),
        ],
        out_specs=(
            pl.BlockSpec((_G, 1), lambda s: (0, 0)),
            pl.BlockSpec((_G, 1), lambda s: (0, 0)),
            pl.BlockSpec((_G, 1), lambda s: (0, 0)),
        ),
        out_shape=out_shapes,
        scratch_shapes=[
            pltpu.VMEM((_G, 1), jnp.float32),
            pltpu.VMEM((_G, 1), jnp.float32),
            pltpu.VMEM((_G, 1), jnp.int32),
            pltpu.VMEM((_G, 1), jnp.float32),
            pltpu.VMEM((_G, 1), jnp.int32),
        ],
        compiler_params=pltpu.CompilerParams(
            dimension_semantics=("arbitrary",),
        ),
    )(X, b2d, W0, b0r, W1, b1r, wfr, bfr)
    return (p[:, 0], act[:, 0], a[:, 0])
